# Initial kernel scaffold; baseline (speedup 1.0000x reference)
import jax
import jax.numpy as jnp
from jax.experimental import pallas as pl


def kernel(input, min_val, delta):
    raise NotImplementedError("write your pallas kernel here")


# speedup vs baseline: 1.0874x; 1.0874x over previous
import jax
import jax.numpy as jnp
from jax.experimental import pallas as pl
from jax.experimental.pallas import tpu as pltpu

jax.config.update("jax_enable_x64", True)

_N_TOKENS = 1024


def _body(scal_ref, x_ref, out_ref):
    min_val = scal_ref[0, 0]
    delta = scal_ref[0, 1]
    y = (x_ref[...] - min_val) / delta
    y = jnp.minimum(jnp.maximum(y, 0.0), jnp.float32(_N_TOKENS - 1))
    out_ref[...] = y.astype(jnp.int32)


def kernel(input, min_val, delta):
    m, n = input.shape
    bm = 256
    grid = (m // bm,)
    with jax.enable_x64(False):
        scal = jnp.stack([min_val.astype(jnp.float32),
                          delta.astype(jnp.float32)]).reshape(1, 2)
        out = pl.pallas_call(
            _body,
            grid=grid,
            in_specs=[
                pl.BlockSpec(memory_space=pltpu.SMEM),
                pl.BlockSpec((bm, n), lambda i: (i, 0)),
            ],
            out_specs=pl.BlockSpec((bm, n), lambda i: (i, 0)),
            out_shape=jax.ShapeDtypeStruct((m, n), jnp.int32),
        )(scal, input)
    return out.astype(jnp.int64)
